# baseline (speedup 1.0000x reference)
import functools
import math

import jax
import jax.numpy as jnp
from jax import lax
from jax.experimental import pallas as pl
from jax.experimental.pallas import tpu as pltpu
from jax.experimental.pallas import tpu_sc as plsc

D = 64
BLANES = 32
NW = 32
CH = 128
NBUF = 8
LOOKAHEAD = 6
SCALE = math.sqrt(D)


def kernel(x, table):
    B0, S = x.shape
    B = B0 * S
    n_chunks = B // (NW * CH)
    assert B % (NW * CH) == 0 and n_chunks % NBUF == 0

    idx2d = x.reshape(B // CH, CH).astype(jnp.int32)
    t16 = table.astype(jnp.bfloat16)
    mesh = plsc.VectorSubcoreMesh(core_axis_name="c", subcore_axis_name="s")

    @functools.partial(
        pl.kernel,
        mesh=mesh,
        out_type=jax.ShapeDtypeStruct((B // CH, CH, D), jnp.bfloat16),
        compiler_params=pltpu.CompilerParams(use_tc_tiling_on_sc=False),
        scratch_types=[
            pltpu.VMEM((n_chunks, CH), jnp.int32),
            pltpu.VMEM((NBUF, CH, D), jnp.bfloat16),
            pltpu.SemaphoreType.DMA((NBUF,)),
            pltpu.SemaphoreType.DMA((NBUF,)),
        ],
    )
    def emb_kernel(idx_hbm, table_hbm, out_hbm, idx_v, rows_v, gsem, ssem):
        wid = lax.axis_index("s") * 2 + lax.axis_index("c")
        chunk0 = wid * n_chunks
        pltpu.sync_copy(idx_hbm.at[pl.ds(chunk0, n_chunks)], idx_v)

        for c in range(LOOKAHEAD):
            pltpu.async_copy(
                table_hbm.at[idx_v.at[c]], rows_v.at[c], gsem.at[c])

        def group(g, carry):
            for b in range(NBUF):
                c = g * NBUF + b
                pltpu.make_async_copy(
                    table_hbm.at[pl.ds(0, CH)], rows_v.at[b], gsem.at[b]
                ).wait()

                @plsc.parallel_loop(0, CH, unroll=8)
                def _(r):
                    for j in range(D // BLANES):
                        sl = pl.ds(j * BLANES, BLANES)
                        rows_v[b, r, sl] = rows_v[b, r, sl] * SCALE

                pltpu.async_copy(
                    rows_v.at[b], out_hbm.at[chunk0 + c], ssem.at[b])

                nb = (b + LOOKAHEAD) % NBUF
                nc = c + LOOKAHEAD

                @pl.when(nc < n_chunks)
                def _():
                    @pl.when(c >= NBUF - LOOKAHEAD)
                    def _():
                        pltpu.make_async_copy(
                            table_hbm.at[pl.ds(0, CH)], rows_v.at[nb],
                            ssem.at[nb],
                        ).wait()

                    pltpu.async_copy(
                        table_hbm.at[idx_v.at[nc]], rows_v.at[nb],
                        gsem.at[nb],
                    )
            return carry

        lax.fori_loop(0, n_chunks // NBUF, group, 0)

        for b in range(NBUF):
            pltpu.make_async_copy(
                table_hbm.at[pl.ds(0, CH)], rows_v.at[b], ssem.at[b]
            ).wait()

    out = emb_kernel(idx2d, t16)
    return out.reshape(B0, S, D).astype(jnp.float32)


# speedup vs baseline: 1.4483x; 1.4483x over previous
import functools
import math

import jax
import jax.numpy as jnp
from jax import lax
from jax.experimental import pallas as pl
from jax.experimental.pallas import tpu as pltpu
from jax.experimental.pallas import tpu_sc as plsc

D = 64
LANES = 16
NW = 32
CH = 128
NBUF = 8
LOOKAHEAD = 6
SCALE = math.sqrt(D)


def kernel(x, table):
    B0, S = x.shape
    B = B0 * S
    n_chunks = B // (NW * CH)
    assert B % (NW * CH) == 0 and n_chunks % NBUF == 0

    idx2d = x.reshape(B // CH, CH).astype(jnp.int32)
    mesh = plsc.VectorSubcoreMesh(core_axis_name="c", subcore_axis_name="s")

    @functools.partial(
        pl.kernel,
        mesh=mesh,
        out_type=jax.ShapeDtypeStruct((B // CH, CH, D), jnp.float32),
        compiler_params=pltpu.CompilerParams(use_tc_tiling_on_sc=False),
        scratch_types=[
            pltpu.VMEM((n_chunks, CH), jnp.int32),
            pltpu.VMEM((NBUF, CH, D), jnp.float32),
            pltpu.SemaphoreType.DMA((NBUF,)),
            pltpu.SemaphoreType.DMA((NBUF,)),
        ],
    )
    def emb_kernel(idx_hbm, table_hbm, out_hbm, idx_v, rows_v, gsem, ssem):
        wid = lax.axis_index("s") * 2 + lax.axis_index("c")
        chunk0 = wid * n_chunks
        pltpu.sync_copy(idx_hbm.at[pl.ds(chunk0, n_chunks)], idx_v)

        for c in range(LOOKAHEAD):
            pltpu.async_copy(
                table_hbm.at[idx_v.at[c]], rows_v.at[c], gsem.at[c])

        def group(g, carry):
            for b in range(NBUF):
                c = g * NBUF + b
                pltpu.make_async_copy(
                    table_hbm.at[pl.ds(0, CH)], rows_v.at[b], gsem.at[b]
                ).wait()

                @plsc.parallel_loop(0, CH, unroll=8)
                def _(r):
                    for j in range(D // LANES):
                        sl = pl.ds(j * LANES, LANES)
                        rows_v[b, r, sl] = rows_v[b, r, sl] * SCALE

                pltpu.async_copy(
                    rows_v.at[b], out_hbm.at[chunk0 + c], ssem.at[b])

                nb = (b + LOOKAHEAD) % NBUF
                nc = c + LOOKAHEAD

                @pl.when(nc < n_chunks)
                def _():
                    @pl.when(c >= NBUF - LOOKAHEAD)
                    def _():
                        pltpu.make_async_copy(
                            table_hbm.at[pl.ds(0, CH)], rows_v.at[nb],
                            ssem.at[nb],
                        ).wait()

                    pltpu.async_copy(
                        table_hbm.at[idx_v.at[nc]], rows_v.at[nb],
                        gsem.at[nb],
                    )
            return carry

        lax.fori_loop(0, n_chunks // NBUF, group, 0)

        for b in range(NBUF):
            pltpu.make_async_copy(
                table_hbm.at[pl.ds(0, CH)], rows_v.at[b], ssem.at[b]
            ).wait()

    out = emb_kernel(idx2d, table)
    return out.reshape(B0, S, D)
